# baseline (speedup 1.0000x reference)
import dataclasses
import functools

import jax
import jax.numpy as jnp
from jax import lax
from jax.experimental import pallas as pl
from jax.experimental.pallas import tpu as pltpu
from jax.experimental.pallas import tpu_sc as plsc

_LANES = 128
_L = 16
_NC, _NS = 2, 16
_NW = _NC * _NS
_CH = 2


def _sc_expand_body(idxf, tbl_hbm, out_hbm, idx_v, buf0, buf1, tbl_v,
                    sem0, sem1, *, batch: int, seq: int, dim: int):
    rows_per_w = batch // _NW
    n_chunks = rows_per_w // _CH
    n_groups = (seq + _L - 1) // _L
    tail = seq - (n_groups - 1) * _L

    wid = lax.axis_index("c") * _NS + lax.axis_index("s")
    base = wid * rows_per_w
    pltpu.sync_copy(tbl_hbm, tbl_v)

    iota = lax.iota(jnp.int32, _L)
    tail_mask = iota < tail
    svecs = [iota + _L * g for g in range(n_groups)]
    dsplats = [jnp.full((_L,), d, jnp.int32) for d in range(dim)]
    rsplats = [jnp.full((_L,), r, jnp.int32) for r in range(_CH)]
    bufs, sems = [buf0, buf1], [sem0, sem1]

    @pl.loop(0, n_chunks, step=2)
    def _(c0):
        for par in range(2):
            c = c0 + par
            buf, sem = bufs[par], sems[par]

            @pl.when(c >= 2)
            def _():
                pltpu.make_async_copy(
                    buf, out_hbm.at[pl.ds(base, _CH)], sem).wait()

            row0 = base + c * _CH
            pltpu.sync_copy(idxf.at[pl.ds(row0 * seq, _CH * seq)],
                            idx_v.at[pl.ds(0, _CH * seq)])
            for r in range(_CH):
                for g in range(n_groups):
                    iv = idx_v[pl.ds(r * seq + g * _L, _L)]
                    if g == n_groups - 1:
                        iv = jnp.where(tail_mask, iv, 0)
                        mask = tail_mask
                    else:
                        mask = None
                    for d in range(dim):
                        vals = plsc.load_gather(tbl_v, [iv, dsplats[d]])
                        plsc.store_scatter(buf,
                                           [rsplats[r], svecs[g], dsplats[d]],
                                           vals, mask=mask)
            pltpu.async_copy(buf, out_hbm.at[pl.ds(row0, _CH)], sem)

    for par in range(2):
        pltpu.make_async_copy(
            bufs[par], out_hbm.at[pl.ds(base, _CH)], sems[par]).wait()


def _pooled_body(idx_ref, rep_ref, src_ref, w_ref, b_ref, ft_ref, pooled_ref,
                 *, dim: int):
    bt = idx_ref.shape[0]
    idx_bf = idx_ref[...].astype(jnp.bfloat16)
    rep = jnp.dot(idx_bf, rep_ref[...], preferred_element_type=jnp.float32)
    repi = rep.astype(jnp.int32)
    kcol = jax.lax.broadcasted_iota(jnp.int32, (bt, _LANES), 1)
    kcol = kcol - dim * (kcol // dim)
    lookup = repi * dim + kcol
    src = jnp.broadcast_to(src_ref[...], (bt, _LANES))
    vals = jnp.take_along_axis(src, lookup, axis=1)
    ft = vals[:, :dim]
    ft_ref[...] = ft
    pooled = jnp.dot(ft, w_ref[...], preferred_element_type=jnp.float32)
    pooled_ref[...] = jnp.tanh(pooled + b_ref[...])


def kernel(inputs, table, W, b):
    batch, seq = inputs.shape
    vocab, dim = table.shape

    mesh = plsc.VectorSubcoreMesh(core_axis_name="c", subcore_axis_name="s")
    cp = pltpu.CompilerParams()
    if "needs_layout_passes" in pltpu.CompilerParams.__dataclass_fields__:
        cp = dataclasses.replace(cp, needs_layout_passes=False)
    sc_call = pl.kernel(
        functools.partial(_sc_expand_body, batch=batch, seq=seq, dim=dim),
        compiler_params=cp,
        mesh=mesh,
        out_type=jax.ShapeDtypeStruct((batch, seq, dim), jnp.float32),
        scratch_types=[
            pltpu.VMEM((_CH * seq + _L,), jnp.int32),
            pltpu.VMEM((_CH, seq, dim), jnp.float32),
            pltpu.VMEM((_CH, seq, dim), jnp.float32),
            pltpu.VMEM((vocab, dim), jnp.float32),
            pltpu.SemaphoreType.DMA,
            pltpu.SemaphoreType.DMA,
        ],
    )
    seq_out = sc_call(inputs.reshape(-1), table)

    block_b = 2048
    j = jnp.arange(_LANES, dtype=jnp.int32)
    rep_mat = (j[None, :] < dim).astype(jnp.bfloat16)
    src_row = jnp.pad(table.reshape(-1), (0, _LANES - vocab * dim))[None, :]
    _, pooled = pl.pallas_call(
        functools.partial(_pooled_body, dim=dim),
        grid=(batch // block_b,),
        in_specs=[
            pl.BlockSpec((block_b, 1), lambda i: (i, 0)),
            pl.BlockSpec((1, _LANES), lambda i: (0, 0)),
            pl.BlockSpec((1, _LANES), lambda i: (0, 0)),
            pl.BlockSpec((dim, dim), lambda i: (0, 0)),
            pl.BlockSpec((1, dim), lambda i: (0, 0)),
        ],
        out_specs=[
            pl.BlockSpec((block_b, dim), lambda i: (i, 0)),
            pl.BlockSpec((block_b, dim), lambda i: (i, 0)),
        ],
        out_shape=[
            jax.ShapeDtypeStruct((batch, dim), jnp.float32),
            jax.ShapeDtypeStruct((batch, dim), jnp.float32),
        ],
    )(inputs[:, 0:1], rep_mat, src_row, W, b[None, :])
    return seq_out, pooled


# speedup vs baseline: 1.0003x; 1.0003x over previous
import dataclasses
import functools

import jax
import jax.numpy as jnp
from jax import lax
from jax.experimental import pallas as pl
from jax.experimental.pallas import tpu as pltpu
from jax.experimental.pallas import tpu_sc as plsc

_LANES = 128
_L = 16
_NC, _NS = 2, 16
_NW = _NC * _NS
_CH = 2


def _sc_expand_body(idxf, tbl_hbm, out_hbm, idx_v, buf0, buf1, tbl_v,
                    sem0, sem1, *, batch: int, seq: int, dim: int):
    rows_per_w = batch // _NW
    n_chunks = rows_per_w // _CH
    n_groups = (seq + _L - 1) // _L
    tail = seq - (n_groups - 1) * _L

    wid = lax.axis_index("c") * _NS + lax.axis_index("s")
    base = wid * rows_per_w
    pltpu.sync_copy(tbl_hbm, tbl_v)

    iota = lax.iota(jnp.int32, _L)
    tail_mask = iota < tail
    svecs = [iota + _L * g for g in range(n_groups)]
    dsplats = [jnp.full((_L,), d, jnp.int32) for d in range(dim)]
    rsplats = [jnp.full((_L,), r, jnp.int32) for r in range(_CH)]
    bufs, sems = [buf0, buf1], [sem0, sem1]

    @pl.loop(0, n_chunks, step=2)
    def _(c0):
        for par in range(2):
            c = c0 + par
            buf, sem = bufs[par], sems[par]

            @pl.when(c >= 2)
            def _():
                pltpu.make_async_copy(
                    buf, out_hbm.at[pl.ds(base, _CH)], sem).wait()

            row0 = base + c * _CH
            pltpu.sync_copy(idxf.at[pl.ds(row0 * seq, _CH * seq)],
                            idx_v.at[pl.ds(0, _CH * seq)])
            for r in range(_CH):
                for g in range(n_groups):
                    iv = idx_v[pl.ds(r * seq + g * _L, _L)]
                    if g == n_groups - 1:
                        iv = jnp.where(tail_mask, iv, 0)
                        mask = tail_mask
                    else:
                        mask = None
                    for d in range(dim):
                        vals = plsc.load_gather(tbl_v, [iv, dsplats[d]])
                        plsc.store_scatter(buf,
                                           [rsplats[r], svecs[g], dsplats[d]],
                                           vals, mask=mask)
            pltpu.async_copy(buf, out_hbm.at[pl.ds(row0, _CH)], sem)

    for par in range(2):
        pltpu.make_async_copy(
            bufs[par], out_hbm.at[pl.ds(base, _CH)], sems[par]).wait()


def _pooled_body(idx_ref, rep_ref, src_ref, w_ref, b_ref, ft_ref, pooled_ref,
                 *, dim: int):
    bt = idx_ref.shape[0]
    idx_bf = idx_ref[...].astype(jnp.bfloat16)
    rep = jnp.dot(idx_bf, rep_ref[...], preferred_element_type=jnp.float32)
    repi = rep.astype(jnp.int32)
    kcol = jax.lax.broadcasted_iota(jnp.int32, (bt, _LANES), 1)
    kcol = kcol - dim * (kcol // dim)
    lookup = repi * dim + kcol
    src = jnp.broadcast_to(src_ref[...], (bt, _LANES))
    vals = jnp.take_along_axis(src, lookup, axis=1)
    ft = vals[:, :dim]
    ft_ref[...] = ft
    pooled = jnp.dot(ft, w_ref[...], preferred_element_type=jnp.float32)
    pooled_ref[...] = jnp.tanh(pooled + b_ref[...])


def kernel(inputs, table, W, b):
    batch, seq = inputs.shape
    vocab, dim = table.shape

    mesh = plsc.VectorSubcoreMesh(core_axis_name="c", subcore_axis_name="s")
    cp = pltpu.CompilerParams(use_tc_tiling_on_sc=True)
    if "needs_layout_passes" in pltpu.CompilerParams.__dataclass_fields__:
        cp = dataclasses.replace(cp, needs_layout_passes=False)
    sc_call = pl.kernel(
        functools.partial(_sc_expand_body, batch=batch, seq=seq, dim=dim),
        compiler_params=cp,
        mesh=mesh,
        out_type=jax.ShapeDtypeStruct((batch, seq, dim), jnp.float32),
        scratch_types=[
            pltpu.VMEM((_CH * seq + _L,), jnp.int32),
            pltpu.VMEM((_CH, seq, dim), jnp.float32),
            pltpu.VMEM((_CH, seq, dim), jnp.float32),
            pltpu.VMEM((vocab, dim), jnp.float32),
            pltpu.SemaphoreType.DMA,
            pltpu.SemaphoreType.DMA,
        ],
    )
    seq_out = sc_call(inputs.reshape(-1), table)

    block_b = 2048
    j = jnp.arange(_LANES, dtype=jnp.int32)
    rep_mat = (j[None, :] < dim).astype(jnp.bfloat16)
    src_row = jnp.pad(table.reshape(-1), (0, _LANES - vocab * dim))[None, :]
    _, pooled = pl.pallas_call(
        functools.partial(_pooled_body, dim=dim),
        grid=(batch // block_b,),
        in_specs=[
            pl.BlockSpec((block_b, 1), lambda i: (i, 0)),
            pl.BlockSpec((1, _LANES), lambda i: (0, 0)),
            pl.BlockSpec((1, _LANES), lambda i: (0, 0)),
            pl.BlockSpec((dim, dim), lambda i: (0, 0)),
            pl.BlockSpec((1, dim), lambda i: (0, 0)),
        ],
        out_specs=[
            pl.BlockSpec((block_b, dim), lambda i: (i, 0)),
            pl.BlockSpec((block_b, dim), lambda i: (i, 0)),
        ],
        out_shape=[
            jax.ShapeDtypeStruct((batch, dim), jnp.float32),
            jax.ShapeDtypeStruct((batch, dim), jnp.float32),
        ],
    )(inputs[:, 0:1], rep_mat, src_row, W, b[None, :])
    return seq_out, pooled


# speedup vs baseline: 3.0773x; 3.0764x over previous
import functools

import jax
import jax.numpy as jnp
from jax.experimental import pallas as pl

_LANES = 128


def _expand_body(idx_ref, rep_ref, src_ref, w_ref, b_ref, out_ref, pooled_ref,
                 *, seq: int, dim: int):
    flat = seq * dim
    flatp = rep_ref.shape[1]
    bt = idx_ref.shape[0]
    idx_bf = idx_ref[...].astype(jnp.bfloat16)
    rep = jnp.dot(idx_bf, rep_ref[...], preferred_element_type=jnp.float32)
    repi = rep.astype(jnp.int32)
    src = jnp.broadcast_to(src_ref[...], (bt, _LANES))
    ft = None
    for j0 in range(0, flatp, _LANES):
        kcol = jax.lax.broadcasted_iota(jnp.int32, (bt, _LANES), 1) + j0
        kcol = kcol - dim * (kcol // dim)
        lookup = repi[:, j0:j0 + _LANES] * dim + kcol
        vals = jnp.take_along_axis(src, lookup, axis=1)
        w = min(_LANES, flat - j0)
        out_ref[:, j0:j0 + w] = vals[:, :w]
        if j0 == 0:
            ft = vals[:, :dim]
    pooled = jnp.dot(ft, w_ref[...], preferred_element_type=jnp.float32)
    pooled_ref[...] = jnp.tanh(pooled + b_ref[...])


def kernel(inputs, table, W, b):
    batch, seq = inputs.shape
    vocab, dim = table.shape
    flat = seq * dim
    flatp = ((flat + _LANES - 1) // _LANES) * _LANES
    block_b = 512

    j = jnp.arange(flatp, dtype=jnp.int32)
    s = jnp.arange(seq, dtype=jnp.int32)
    rep_mat = (s[:, None] == (j[None, :] // dim)).astype(jnp.bfloat16)
    src_row = jnp.pad(table.reshape(-1), (0, _LANES - vocab * dim))[None, :]

    grid = (batch // block_b,)
    out_flat, pooled = pl.pallas_call(
        functools.partial(_expand_body, seq=seq, dim=dim),
        grid=grid,
        in_specs=[
            pl.BlockSpec((block_b, seq), lambda i: (i, 0)),
            pl.BlockSpec((seq, flatp), lambda i: (0, 0)),
            pl.BlockSpec((1, _LANES), lambda i: (0, 0)),
            pl.BlockSpec((dim, dim), lambda i: (0, 0)),
            pl.BlockSpec((1, dim), lambda i: (0, 0)),
        ],
        out_specs=[
            pl.BlockSpec((block_b, flat), lambda i: (i, 0)),
            pl.BlockSpec((block_b, dim), lambda i: (i, 0)),
        ],
        out_shape=[
            jax.ShapeDtypeStruct((batch, flat), jnp.float32),
            jax.ShapeDtypeStruct((batch, dim), jnp.float32),
        ],
    )(inputs, rep_mat, src_row, W, b[None, :])
    return out_flat.reshape(batch, seq, dim), pooled
